# baseline (speedup 1.0000x reference)
import jax
import jax.numpy as jnp
from jax.experimental import pallas as pl


def _pool_conv_broadcast_kernel(x_ref, w_ref, b_ref, o_ref):
    x = x_ref[0]
    m = jnp.mean(x, axis=1, keepdims=True)
    pooled = jnp.dot(w_ref[...], m,
                     preferred_element_type=jnp.float32) + b_ref[...]
    o_ref[0] = jnp.broadcast_to(pooled, o_ref.shape[1:])


def kernel(features, depth, W, b):
    del depth
    N, C, H, Wd = features.shape
    O = W.shape[0]
    x = features.reshape(N, C, H * Wd)
    w2 = W.reshape(O, C)
    b2 = b.reshape(O, 1)
    out = pl.pallas_call(
        _pool_conv_broadcast_kernel,
        grid=(N,),
        in_specs=[
            pl.BlockSpec((1, C, H * Wd), lambda i: (i, 0, 0)),
            pl.BlockSpec((O, C), lambda i: (0, 0)),
            pl.BlockSpec((O, 1), lambda i: (0, 0)),
        ],
        out_specs=pl.BlockSpec((1, O, H * Wd), lambda i: (i, 0, 0)),
        out_shape=jax.ShapeDtypeStruct((N, O, H * Wd), jnp.float32),
    )(x, w2, b2)
    return out.reshape(N, O, H, Wd)
